# Initial kernel scaffold; baseline (speedup 1.0000x reference)
import jax
import jax.numpy as jnp
from jax.experimental import pallas as pl


def kernel(query_features, support_features, support_y, node_W1, node_b1, node_W2, node_b2, msg_W1, msg_b1, msg_W2, msg_b2, rel_W1, rel_b1, rel_W2, rel_b2):
    raise NotImplementedError("write your pallas kernel here")


# speedup vs baseline: 2.7630x; 2.7630x over previous
import jax
import jax.numpy as jnp
from jax.experimental import pallas as pl
from jax.experimental.pallas import tpu as pltpu

_NQ, _NS, _E, _H = 256, 128, 256, 256
_BI = 32
_BQ = 32


def _dot(a, b):
    return jax.lax.dot_general(a, b, (((1,), (0,)), ((), ())),
                               preferred_element_type=jnp.float32)


def _body(qf, sf, y_row, y_col, nW1, nb1, nW2, nb2,
          m1a0, m1b0, mb10, mW20, mb20,
          m1a1, m1b1, mb11, mW21, mb21,
          ra, rb, rb1, rw2, rb2,
          out, s_ref, snew_ref):
    s = _dot(jnp.maximum(_dot(sf[...], nW1[...]) + nb1[...], 0.0), nW2[...]) + nb2[...]
    s_ref[...] = s
    q = _dot(jnp.maximum(_dot(qf[...], nW1[...]) + nb1[...], 0.0), nW2[...]) + nb2[...]

    same_f = (y_col[...] == y_row[...]).astype(jnp.float32)
    cc = jnp.sum(same_f, axis=1, keepdims=True)

    for (w1a, w1b, b1, W2, b2) in ((m1a0, m1b0, mb10, mW20, mb20),
                                   (m1a1, m1b1, mb11, mW21, mb21)):
        s = s_ref[...]
        A = _dot(s, w1a[...]) + b1[...]
        B = _dot(s, w1b[...])
        for r0 in range(0, _NS, _BI):
            s_blk = s[r0:r0 + _BI]
            ndiff = jnp.sum((s_blk[:, None, :] != s[None, :, :]).astype(jnp.float32),
                            axis=-1)
            valid = same_f[r0:r0 + _BI] * (ndiff > 0.0).astype(jnp.float32)
            T = jnp.maximum(A[r0:r0 + _BI][:, None, :] + B[None, :, :], 0.0)
            R = jnp.sum(T * valid[:, :, None], axis=1)
            nv = jnp.sum(valid, axis=1, keepdims=True)
            agg = _dot(R / jnp.maximum(nv, 1.0), W2[...]) + b2[...]
            upd = (cc[r0:r0 + _BI] > 1.0) & (nv > 0.0)
            snew_ref[r0:r0 + _BI, :] = jnp.where(upd, s_blk + agg, s_blk)
        s_ref[...] = snew_ref[...]

    s = s_ref[...]
    qA = _dot(q, ra[...]) + rb1[...]
    sB = _dot(s, rb[...])
    w2 = rw2[...]
    bias = rb2[0, 0]
    for r0 in range(0, _NQ, _BQ):
        T = jnp.maximum(qA[r0:r0 + _BQ][:, None, :] + sB[None, :, :], 0.0)
        out[r0:r0 + _BQ, :] = jnp.sum(T * w2[None, :, :], axis=-1) + bias


@jax.jit
def kernel(query_features, support_features, support_y,
           node_W1, node_b1, node_W2, node_b2,
           msg_W1, msg_b1, msg_W2, msg_b2,
           rel_W1, rel_b1, rel_W2, rel_b2):
    y_row = support_y.reshape(1, _NS)
    y_col = support_y.reshape(_NS, 1)
    args = (
        query_features, support_features, y_row, y_col,
        node_W1, node_b1.reshape(1, _H), node_W2, node_b2.reshape(1, _H),
        msg_W1[0, :_H], msg_W1[0, _H:], msg_b1[0].reshape(1, _H),
        msg_W2[0], msg_b2[0].reshape(1, _H),
        msg_W1[1, :_H], msg_W1[1, _H:], msg_b1[1].reshape(1, _H),
        msg_W2[1], msg_b2[1].reshape(1, _H),
        rel_W1[:_H], rel_W1[_H:], rel_b1.reshape(1, _H),
        rel_W2.reshape(1, _H), rel_b2.reshape(1, 1),
    )
    return pl.pallas_call(
        _body,
        out_shape=jax.ShapeDtypeStruct((_NQ, _NS), jnp.float32),
        in_specs=[pl.BlockSpec(memory_space=pltpu.VMEM) for _ in args],
        out_specs=pl.BlockSpec(memory_space=pltpu.VMEM),
        scratch_shapes=[pltpu.VMEM((_NS, _H), jnp.float32),
                        pltpu.VMEM((_NS, _H), jnp.float32)],
    )(*args)


# speedup vs baseline: 2.9604x; 1.0714x over previous
import jax
import jax.numpy as jnp
from jax.experimental import pallas as pl
from jax.experimental.pallas import tpu as pltpu

_NQ, _NS, _E, _H = 256, 128, 256, 256
_BI = 32
_BQ = 32


def _dot(a, b):
    return jax.lax.dot_general(a, b, (((1,), (0,)), ((), ())),
                               preferred_element_type=jnp.float32)


def _body(qf, sf, y_row, y_col, nW1, nb1, nW2, nb2,
          m1a0, m1b0, mb10, mW20, mb20,
          m1a1, m1b1, mb11, mW21, mb21,
          ra, rb, rb1, rw2, rb2,
          out, s_ref, snew_ref):
    s = _dot(jnp.maximum(_dot(sf[...], nW1[...]) + nb1[...], 0.0), nW2[...]) + nb2[...]
    s_ref[...] = s
    q = _dot(jnp.maximum(_dot(qf[...], nW1[...]) + nb1[...], 0.0), nW2[...]) + nb2[...]

    same_f = (y_col[...] == y_row[...]).astype(jnp.float32)
    cc = jnp.sum(same_f, axis=1, keepdims=True)

    for (w1a, w1b, b1, W2, b2) in ((m1a0, m1b0, mb10, mW20, mb20),
                                   (m1a1, m1b1, mb11, mW21, mb21)):
        s = s_ref[...]
        A = _dot(s, w1a[...]) + b1[...]
        B = _dot(s, w1b[...])
        for r0 in range(0, _NS, _BI):
            s_blk = s[r0:r0 + _BI]
            neq = jnp.any(s_blk[:, None, :] != s[None, :, :], axis=-1)
            valid = same_f[r0:r0 + _BI] * neq.astype(jnp.float32)
            T = jnp.maximum(A[r0:r0 + _BI][:, None, :] + B[None, :, :], 0.0)
            R = jax.lax.dot_general(valid, T, (((1,), (1,)), ((0,), (0,))),
                                    preferred_element_type=jnp.float32)
            nv = jnp.sum(valid, axis=1, keepdims=True)
            agg = _dot(R / jnp.maximum(nv, 1.0), W2[...]) + b2[...]
            upd = (cc[r0:r0 + _BI] > 1.0) & (nv > 0.0)
            snew_ref[r0:r0 + _BI, :] = jnp.where(upd, s_blk + agg, s_blk)
        s_ref[...] = snew_ref[...]

    s = s_ref[...]
    qA = _dot(q, ra[...]) + rb1[...]
    sB = _dot(s, rb[...])
    w2 = rw2[...]
    bias = rb2[0, 0]
    for r0 in range(0, _NQ, _BQ):
        T = jnp.maximum(qA[r0:r0 + _BQ][:, None, :] + sB[None, :, :], 0.0)
        out[r0:r0 + _BQ, :] = jnp.sum(T * w2[None, :, :], axis=-1) + bias


@jax.jit
def kernel(query_features, support_features, support_y,
           node_W1, node_b1, node_W2, node_b2,
           msg_W1, msg_b1, msg_W2, msg_b2,
           rel_W1, rel_b1, rel_W2, rel_b2):
    y_row = support_y.reshape(1, _NS)
    y_col = support_y.reshape(_NS, 1)
    args = (
        query_features, support_features, y_row, y_col,
        node_W1, node_b1.reshape(1, _H), node_W2, node_b2.reshape(1, _H),
        msg_W1[0, :_H], msg_W1[0, _H:], msg_b1[0].reshape(1, _H),
        msg_W2[0], msg_b2[0].reshape(1, _H),
        msg_W1[1, :_H], msg_W1[1, _H:], msg_b1[1].reshape(1, _H),
        msg_W2[1], msg_b2[1].reshape(1, _H),
        rel_W1[:_H], rel_W1[_H:], rel_b1.reshape(1, _H),
        rel_W2.reshape(1, _H), rel_b2.reshape(1, 1),
    )
    return pl.pallas_call(
        _body,
        out_shape=jax.ShapeDtypeStruct((_NQ, _NS), jnp.float32),
        in_specs=[pl.BlockSpec(memory_space=pltpu.VMEM) for _ in args],
        out_specs=pl.BlockSpec(memory_space=pltpu.VMEM),
        scratch_shapes=[pltpu.VMEM((_NS, _H), jnp.float32),
                        pltpu.VMEM((_NS, _H), jnp.float32)],
    )(*args)
